# baseline (device time: 16716 ns/iter reference)
import jax
import jax.numpy as jnp
from jax import lax
from jax.experimental import pallas as pl
from jax.experimental.pallas import tpu as pltpu

N_DEV = 4


def kernel(x, Wq, Wo, K_ext, V_ext):
    B, Sq, D = x.shape
    Dq = Wq.shape[1]
    Dh = K_ext.shape[3]
    Skv = K_ext.shape[1]
    Hq_local = Dq // Dh
    GQA = 4
    Dout = Wo.shape[1]
    M = B * Sq

    def body(x_ref, wq_ref, wo_ref, k_ref, v_ref, out_ref,
             comm_ref, send_sems, recv_sems):
        my_i = lax.axis_index("i")

        barrier_sem = pltpu.get_barrier_semaphore()
        for d in range(1, N_DEV):
            peer = lax.rem(my_i + d, N_DEV)
            pl.semaphore_signal(
                barrier_sem, inc=1,
                device_id=(peer,), device_id_type=pl.DeviceIdType.MESH,
            )
        pl.semaphore_wait(barrier_sem, N_DEV - 1)

        xv = x_ref[:].reshape(M, D).astype(jnp.bfloat16)
        wq = wq_ref[:].astype(jnp.bfloat16)
        q2 = (lax.dot(xv, wq, preferred_element_type=jnp.float32)
              * 0.125).astype(jnp.bfloat16)

        kv_base = 2 * my_i
        batch_rows = []
        for b in range(B):
            qb = q2[b * Sq:(b + 1) * Sq, :]
            heads = []
            for g in range(Hq_local // GQA):
                kb = k_ref[b, :, pl.ds(kv_base + g, 1), :].reshape(Skv, Dh)
                vb = v_ref[b, :, pl.ds(kv_base + g, 1), :].reshape(Skv, Dh)
                kb = kb.astype(jnp.bfloat16)
                vb = vb.astype(jnp.bfloat16)
                qg = jnp.concatenate(
                    [qb[:, (g * GQA + hh) * Dh:(g * GQA + hh + 1) * Dh]
                     for hh in range(GQA)], axis=0)
                s = lax.dot_general(
                    qg, kb, (((1,), (1,)), ((), ())),
                    preferred_element_type=jnp.float32,
                )
                m = jnp.max(s, axis=1, keepdims=True)
                p = jnp.exp(s - m)
                l = jnp.sum(p, axis=1, keepdims=True)
                o = lax.dot(p.astype(jnp.bfloat16), vb,
                            preferred_element_type=jnp.float32)
                o = o / l
                heads.extend(o[hh * Sq:(hh + 1) * Sq, :] for hh in range(GQA))
            batch_rows.append(jnp.concatenate(heads, axis=1))
        attn = jnp.concatenate(batch_rows, axis=0)

        wo = wo_ref[:].astype(jnp.bfloat16)
        partial = lax.dot(attn.astype(jnp.bfloat16), wo,
                          preferred_element_type=jnp.float32)

        comm_ref[0, :, :] = partial.astype(jnp.bfloat16)
        rdmas = []
        for d in range(1, N_DEV):
            peer = lax.rem(my_i + d, N_DEV)
            slot = N_DEV - d
            rdma = pltpu.make_async_remote_copy(
                src_ref=comm_ref.at[0],
                dst_ref=comm_ref.at[slot],
                send_sem=send_sems.at[d - 1],
                recv_sem=recv_sems.at[slot - 1],
                device_id=(peer,),
                device_id_type=pl.DeviceIdType.MESH,
            )
            rdma.start()
            rdmas.append(rdma)

        acc = partial
        for s in range(1, N_DEV):
            rdmas[N_DEV - 1 - s].wait_recv()
            acc = acc + comm_ref[s, :, :].astype(jnp.float32)

        for rdma in rdmas:
            rdma.wait_send()

        out_ref[:] = acc.reshape(B, Sq, Dout)

    return pl.pallas_call(
        body,
        out_shape=jax.ShapeDtypeStruct((B, Sq, Dout), jnp.float32),
        in_specs=[pl.BlockSpec(memory_space=pltpu.VMEM)] * 5,
        out_specs=pl.BlockSpec(memory_space=pltpu.VMEM),
        scratch_shapes=[
            pltpu.VMEM((N_DEV, M, Dout), jnp.bfloat16),
            pltpu.SemaphoreType.DMA((N_DEV - 1,)),
            pltpu.SemaphoreType.DMA((N_DEV - 1,)),
        ],
        compiler_params=pltpu.CompilerParams(collective_id=0),
    )(x, Wq, Wo, K_ext, V_ext)
